# baseline (speedup 1.0000x reference)
import jax
import jax.numpy as jnp
from jax.experimental import pallas as pl
from jax.experimental.pallas import tpu as pltpu

_B, _C, _H, _W = 16, 64, 32, 32
_N = _H * _W
_K = 1024
_BB = 2


def _vq_one_batch(xb, cb, cb_aug):
    s = jax.lax.dot_general(-2.0 * cb, xb, (((1,), (0,)), ((), ())),
                            preferred_element_type=jnp.float32)
    x_sqr = jnp.sum(xb * xb, axis=0, keepdims=True)
    cb_sqr = jnp.sum(cb * cb, axis=1, keepdims=True)
    dist = (x_sqr + cb_sqr) + s
    minval = jnp.min(dist, axis=0, keepdims=True)
    mask = dist == minval
    onehot = jnp.where(mask, jnp.float32(1.0), jnp.float32(0.0))
    out_aug = jax.lax.dot_general(cb_aug, onehot, (((0,), (0,)), ((), ())),
                                  preferred_element_type=jnp.float32)
    codes = out_aug[:_C]
    indf = out_aug[_C] * 32.0 + out_aug[_C + 1]
    return codes, indf[None].astype(jnp.int32)


def _vq_kernel(x_ref, cb_ref, codes_ref, ind_ref):
    cb = cb_ref[...]
    kcol = jax.lax.broadcasted_iota(jnp.int32, (_K, 1), 0)
    hi = (kcol >> 5).astype(jnp.float32)
    lo = (kcol & 31).astype(jnp.float32)
    cb_aug = jnp.concatenate([cb, hi, lo], axis=1)
    for j in range(_BB):
        codes, ind = _vq_one_batch(x_ref[j], cb, cb_aug)
        codes_ref[j] = codes
        ind_ref[j] = ind


def kernel(x, codebook):
    x2 = x.reshape(_B, _C, _N)
    codes2, ind2 = pl.pallas_call(
        _vq_kernel,
        grid=(_B // _BB,),
        in_specs=[pl.BlockSpec((_BB, _C, _N), lambda b: (b, 0, 0)),
                  pl.BlockSpec((_K, _C), lambda b: (0, 0))],
        out_specs=[pl.BlockSpec((_BB, _C, _N), lambda b: (b, 0, 0)),
                   pl.BlockSpec((_BB, 1, _N), lambda b: (b, 0, 0))],
        out_shape=[jax.ShapeDtypeStruct((_B, _C, _N), jnp.float32),
                   jax.ShapeDtypeStruct((_B, 1, _N), jnp.int32)],
        compiler_params=pltpu.CompilerParams(
            dimension_semantics=("arbitrary",)),
    )(x2, codebook)
    return codes2.reshape(_B, _C, _H, _W), ind2.reshape(_B, _H, _W)


# speedup vs baseline: 1.0023x; 1.0023x over previous
import jax
import jax.numpy as jnp
from jax.experimental import pallas as pl
from jax.experimental.pallas import tpu as pltpu

_B, _C, _H, _W = 16, 64, 32, 32
_N = _H * _W
_K = 1024
_BB = 4


def _vq_one_batch(xb, cb_m2, cb_aug, cb_sqr):
    s = jax.lax.dot_general(cb_m2, xb, (((1,), (0,)), ((), ())),
                            preferred_element_type=jnp.float32)
    x_sqr = jnp.sum(xb * xb, axis=0, keepdims=True)
    dist = (x_sqr + cb_sqr) + s
    minval = jnp.min(dist, axis=0, keepdims=True)
    mask = dist == minval
    onehot = jnp.where(mask, jnp.float32(1.0), jnp.float32(0.0))
    out_aug = jax.lax.dot_general(cb_aug, onehot, (((0,), (0,)), ((), ())),
                                  preferred_element_type=jnp.float32)
    codes = out_aug[:_C]
    indf = out_aug[_C] * 32.0 + out_aug[_C + 1]
    return codes, indf[None].astype(jnp.int32)


def _vq_kernel(x_ref, cb_ref, codes_ref, ind_ref):
    cb = cb_ref[...]
    kcol = jax.lax.broadcasted_iota(jnp.int32, (_K, 1), 0)
    hi = (kcol >> 5).astype(jnp.float32)
    lo = (kcol & 31).astype(jnp.float32)
    cb_aug = jnp.concatenate([cb, hi, lo], axis=1)
    cb_sqr = jnp.sum(cb * cb, axis=1, keepdims=True)
    cb_m2 = -2.0 * cb
    for j in range(_BB):
        codes, ind = _vq_one_batch(x_ref[j], cb_m2, cb_aug, cb_sqr)
        codes_ref[j] = codes
        ind_ref[j] = ind


def kernel(x, codebook):
    x2 = x.reshape(_B, _C, _N)
    codes2, ind2 = pl.pallas_call(
        _vq_kernel,
        grid=(_B // _BB,),
        in_specs=[pl.BlockSpec((_BB, _C, _N), lambda b: (b, 0, 0)),
                  pl.BlockSpec((_K, _C), lambda b: (0, 0))],
        out_specs=[pl.BlockSpec((_BB, _C, _N), lambda b: (b, 0, 0)),
                   pl.BlockSpec((_BB, 1, _N), lambda b: (b, 0, 0))],
        out_shape=[jax.ShapeDtypeStruct((_B, _C, _N), jnp.float32),
                   jax.ShapeDtypeStruct((_B, 1, _N), jnp.int32)],
        compiler_params=pltpu.CompilerParams(
            dimension_semantics=("arbitrary",)),
    )(x2, codebook)
    return codes2.reshape(_B, _C, _H, _W), ind2.reshape(_B, _H, _W)
